# baseline (speedup 1.0000x reference)
import functools

import jax
import jax.numpy as jnp
from jax import lax
from jax.experimental import pallas as pl
from jax.experimental.pallas import tpu as pltpu
from jax.experimental.pallas import tpu_sc as plsc

_N = 10000
_E = 320000
_D = 128
_G = 128
_OUT = 4

_NC = 2
_NS = 16
_NW = _NC * _NS
_EPW = _E // _NW
_CH = 128
_NSB = 10
_SBC = 8
_EPWP = _NSB * _SBC * _CH
_PADW = _EPWP - _EPW
_NP = 10112
_RPS = _NP // _NS

_BLK = 1000
_NB = _N // _BLK


def _mesh():
  return plsc.VectorSubcoreMesh(core_axis_name="c", subcore_axis_name="s",
                                num_cores=_NC, num_subcores=_NS)


def _make_sc_agg():
  out_type = [jax.ShapeDtypeStruct((_NC, _NP, _D), jnp.float32)]
  scratch = [
      pltpu.VMEM((2, _SBC, _CH), jnp.int32),
      pltpu.VMEM((2, _SBC, _CH), jnp.int32),
      pltpu.VMEM((_CH, _D), jnp.float32),
      pltpu.VMEM((_CH, _D), jnp.float32),
      pltpu.VMEM_SHARED((_NP, _D), jnp.float32),
      pltpu.SemaphoreType.DMA,
      pltpu.SemaphoreType.DMA,
      pltpu.SemaphoreType.DMA,
  ]

  def body(x_hbm, src_hbm, dst_hbm, zeros_hbm, agg_out,
           srcv, dstv, rows0, rows1, aggsh, sem0, sem1, semi):
    rows = (rows0, rows1)
    sems = (sem0, sem1)
    c = lax.axis_index("c")
    s = lax.axis_index("s")
    wid = c * _NS + s
    pltpu.sync_copy(zeros_hbm.at[pl.ds(s * _RPS, _RPS)],
                    aggsh.at[pl.ds(s * _RPS, _RPS)])
    plsc.subcore_barrier()

    pltpu.sync_copy(src_hbm.at[wid, 0], srcv.at[0])
    pltpu.sync_copy(dst_hbm.at[wid, 0], dstv.at[0])

    def superblock(b, carry):
      p = lax.rem(b, 2)
      nxt = jnp.minimum(b + 1, _NSB - 1)
      np_ = lax.rem(b + 1, 2)
      pltpu.async_copy(src_hbm.at[wid, nxt], srcv.at[np_], semi)
      pltpu.async_copy(dst_hbm.at[wid, nxt], dstv.at[np_], semi)

      pltpu.async_copy(x_hbm.at[srcv.at[p, 0]], rows0, sem0)
      for i in range(_SBC):
        if i + 1 < _SBC:
          pltpu.async_copy(x_hbm.at[srcv.at[p, i + 1]],
                           rows[(i + 1) % 2], sems[(i + 1) % 2])
        pltpu.make_async_copy(x_hbm.at[srcv.at[p, 0]],
                              rows[i % 2], sems[i % 2]).wait()
        pltpu.sync_copy(rows[i % 2], aggsh.at[dstv.at[p, i]], add=True)

      pltpu.make_async_copy(src_hbm.at[wid, 0], srcv.at[np_], semi).wait()
      pltpu.make_async_copy(dst_hbm.at[wid, 0], dstv.at[np_], semi).wait()
      return carry

    lax.fori_loop(0, _NSB, superblock, 0)

    plsc.subcore_barrier()
    pltpu.sync_copy(aggsh.at[pl.ds(s * _RPS, _RPS)],
                    agg_out.at[c, pl.ds(s * _RPS, _RPS)])

  return pl.kernel(body, out_type=out_type, mesh=_mesh(),
                   scratch_types=scratch, name="sc_agg")


_EBLK = 4000
_NEB = _E // _EBLK
_HB = _NP // _G


def _hist_body(dst_ref, out_ref):
  i = pl.program_id(0)

  @pl.when(i == 0)
  def _():
    out_ref[...] = jnp.zeros_like(out_ref)

  d = dst_ref[0, 0, :]
  hi = lax.shift_right_logical(d, 7)
  lo = jnp.bitwise_and(d, 127)
  oh_hi = (lax.broadcasted_iota(jnp.int32, (_HB, _EBLK), 0)
           == hi[None, :]).astype(jnp.bfloat16)
  oh_lo = (lax.broadcasted_iota(jnp.int32, (_EBLK, _G), 1)
           == lo[:, None]).astype(jnp.bfloat16)
  out_ref[...] += jnp.dot(oh_hi, oh_lo, preferred_element_type=jnp.float32)


def _hist_tc(dst3):
  return pl.pallas_call(
      _hist_body,
      grid=(_NEB,),
      in_specs=[pl.BlockSpec((1, 1, _EBLK), lambda i: (i, 0, 0))],
      out_specs=pl.BlockSpec((_HB, _G), lambda i: (0, 0)),
      out_shape=jax.ShapeDtypeStruct((_HB, _G), jnp.float32),
  )(dst3)


_sc_cache = {}


def _get_sc(name):
  if name not in _sc_cache:
    _sc_cache[name] = _make_sc_agg()
  return _sc_cache[name]


def _layer_body(aggp_ref, cnt_ref, x_ref, wlt_ref, bl_ref, wrt_ref, h_ref):
  agg = aggp_ref[0] + aggp_ref[1]
  cnt = cnt_ref[...]
  mm = jnp.dot(agg, wlt_ref[...], preferred_element_type=jnp.float32)
  self_mm = jnp.dot(x_ref[...], wrt_ref[...], preferred_element_type=jnp.float32)
  h = mm / jnp.maximum(cnt, 1.0) + bl_ref[...] + self_mm
  h_ref[...] = jnp.maximum(h, 0.0)


def _layer_tc(aggp, cnt, x, wlt, bl, wrt):
  return pl.pallas_call(
      _layer_body,
      grid=(_NB,),
      in_specs=[
          pl.BlockSpec((_NC, _BLK, _D), lambda i: (0, i, 0)),
          pl.BlockSpec((_BLK, 1), lambda i: (i, 0)),
          pl.BlockSpec((_BLK, _D), lambda i: (i, 0)),
          pl.BlockSpec((_D, _D), lambda i: (0, 0)),
          pl.BlockSpec((1, _D), lambda i: (0, 0)),
          pl.BlockSpec((_D, _D), lambda i: (0, 0)),
      ],
      out_specs=pl.BlockSpec((_BLK, _D), lambda i: (i, 0)),
      out_shape=jax.ShapeDtypeStruct((_N, _D), jnp.float32),
  )(aggp, cnt, x, wlt, bl, wrt)


def _final_body(aggp_ref, cnt_ref, h_ref, batch_ref, wlt_ref, bl_ref, wrt_ref,
                wlint_ref, blin_ref, out_ref, pooled_acc, gcnt_acc):
  i = pl.program_id(0)

  @pl.when(i == 0)
  def _():
    pooled_acc[...] = jnp.zeros_like(pooled_acc)
    gcnt_acc[...] = jnp.zeros_like(gcnt_acc)

  agg = aggp_ref[0] + aggp_ref[1]
  cnt = cnt_ref[...]
  mm = jnp.dot(agg, wlt_ref[...], preferred_element_type=jnp.float32)
  self_mm = jnp.dot(h_ref[...], wrt_ref[...], preferred_element_type=jnp.float32)
  h2 = jnp.maximum(mm / jnp.maximum(cnt, 1.0) + bl_ref[...] + self_mm, 0.0)

  b = batch_ref[0, 0, :]
  onehot = (lax.broadcasted_iota(jnp.int32, (_G, _BLK), 0)
            == b[None, :]).astype(jnp.float32)
  pooled_acc[...] += jnp.dot(onehot, h2, preferred_element_type=jnp.float32)
  gcnt_acc[...] += jnp.sum(onehot, axis=1, keepdims=True)

  @pl.when(i == _NB - 1)
  def _():
    pooled = pooled_acc[...] / jnp.maximum(gcnt_acc[...], 1.0)
    out_ref[...] = (jnp.dot(pooled, wlint_ref[...],
                            preferred_element_type=jnp.float32)
                    + blin_ref[...])


def _final_tc(aggp, cnt, h, batch3, wlt, bl, wrt, wlint_pad, blin_pad):
  return pl.pallas_call(
      _final_body,
      grid=(_NB,),
      in_specs=[
          pl.BlockSpec((_NC, _BLK, _D), lambda i: (0, i, 0)),
          pl.BlockSpec((_BLK, 1), lambda i: (i, 0)),
          pl.BlockSpec((_BLK, _D), lambda i: (i, 0)),
          pl.BlockSpec((1, 1, _BLK), lambda i: (i, 0, 0)),
          pl.BlockSpec((_D, _D), lambda i: (0, 0)),
          pl.BlockSpec((1, _D), lambda i: (0, 0)),
          pl.BlockSpec((_D, _D), lambda i: (0, 0)),
          pl.BlockSpec((_D, _D), lambda i: (0, 0)),
          pl.BlockSpec((1, _D), lambda i: (0, 0)),
      ],
      out_specs=pl.BlockSpec((_G, _D), lambda i: (0, 0)),
      out_shape=jax.ShapeDtypeStruct((_G, _D), jnp.float32),
      scratch_shapes=[
          pltpu.VMEM((_G, _D), jnp.float32),
          pltpu.VMEM((_G, 1), jnp.float32),
      ],
  )(aggp, cnt, h, batch3, wlt, bl, wrt, wlint_pad, blin_pad)


def kernel(x, edge_index, batch, Wl1, bl1, Wr1, Wl2, bl2, Wr2, Wlin, blin):
  pad_src = jnp.zeros((_NW, _PADW), jnp.int32)
  pad_dst = jnp.broadcast_to(
      _N + (jnp.arange(_PADW, dtype=jnp.int32) % (_NP - _N)), (_NW, _PADW))
  src = jnp.concatenate([edge_index[0].reshape(_NW, _EPW), pad_src],
                        axis=1).reshape(_NW, _NSB, _SBC, _CH)
  dst = jnp.concatenate([edge_index[1].reshape(_NW, _EPW), pad_dst],
                        axis=1).reshape(_NW, _NSB, _SBC, _CH)
  zeros = jnp.zeros((_NP, _D), jnp.float32)

  hist = _hist_tc(edge_index[1].reshape(_NEB, 1, _EBLK))
  (agg1p,) = _get_sc("agg")(x, src, dst, zeros)
  cnt = hist.reshape(_HB * _G, 1)[:_N]
  h1 = _layer_tc(agg1p, cnt, x, Wl1.T, bl1.reshape(1, _D), Wr1.T)
  (agg2p,) = _get_sc("agg")(h1, src, dst, zeros)

  batch3 = batch.reshape(_NB, 1, _BLK)
  wlint_pad = jnp.pad(Wlin, ((0, _D - _OUT), (0, 0))).T
  blin_pad = jnp.pad(blin, (0, _D - _OUT)).reshape(1, _D)
  out_pad = _final_tc(agg2p, cnt, h1, batch3, Wl2.T, bl2.reshape(1, _D),
                      Wr2.T, wlint_pad, blin_pad)
  return out_pad[:, :_OUT]


# speedup vs baseline: 2.4664x; 2.4664x over previous
import functools

import jax
import jax.numpy as jnp
from jax import lax
from jax.experimental import pallas as pl
from jax.experimental.pallas import tpu as pltpu
from jax.experimental.pallas import tpu_sc as plsc

_N = 10000
_E = 320000
_D = 128
_G = 128
_OUT = 4

_NC = 2
_NS = 16
_NW = _NC * _NS
_EPW = _E // _NW
_CH = 80
_NSB = 25
_SBC = 5
_EPWP = _NSB * _SBC * _CH
_PADW = _EPWP - _EPW
_NP = 10112
_RPS = _NP // _NS

_BLK = 1000
_NB = _N // _BLK


def _mesh():
  return plsc.VectorSubcoreMesh(core_axis_name="c", subcore_axis_name="s",
                                num_cores=_NC, num_subcores=_NS)


def _make_sc_agg():
  out_type = [jax.ShapeDtypeStruct((_NC, _NP, _D), jnp.float32)]
  scratch = [
      pltpu.VMEM((2, _SBC, _CH), jnp.int32),
      pltpu.VMEM((2, _SBC, _CH), jnp.int32),
      pltpu.VMEM((_CH, _D), jnp.float32),
      pltpu.VMEM((_CH, _D), jnp.float32),
      pltpu.VMEM_SHARED((_NP, _D), jnp.float32),
      pltpu.SemaphoreType.DMA,
      pltpu.SemaphoreType.DMA,
      pltpu.SemaphoreType.DMA,
  ]

  def body(x_hbm, src_hbm, dst_hbm, zeros_hbm, agg_out,
           srcv, dstv, rows0, rows1, aggsh, sem0, sem1, semi):
    rows = (rows0, rows1)
    sems = (sem0, sem1)
    c = lax.axis_index("c")
    s = lax.axis_index("s")
    wid = c * _NS + s
    pltpu.sync_copy(zeros_hbm.at[pl.ds(s * _RPS, _RPS)],
                    aggsh.at[pl.ds(s * _RPS, _RPS)])
    plsc.subcore_barrier()

    pltpu.sync_copy(src_hbm.at[wid, 0], srcv.at[0])
    pltpu.sync_copy(dst_hbm.at[wid, 0], dstv.at[0])

    def superblock(b, carry):
      p = lax.rem(b, 2)
      nxt = jnp.minimum(b + 1, _NSB - 1)
      np_ = lax.rem(b + 1, 2)
      pltpu.async_copy(src_hbm.at[wid, nxt], srcv.at[np_], semi)
      pltpu.async_copy(dst_hbm.at[wid, nxt], dstv.at[np_], semi)

      pltpu.async_copy(x_hbm.at[srcv.at[p, 0]], rows0, sem0)
      for i in range(_SBC):
        if i + 1 < _SBC:
          pltpu.async_copy(x_hbm.at[srcv.at[p, i + 1]],
                           rows[(i + 1) % 2], sems[(i + 1) % 2])
        pltpu.make_async_copy(x_hbm.at[srcv.at[p, 0]],
                              rows[i % 2], sems[i % 2]).wait()
        pltpu.sync_copy(rows[i % 2], aggsh.at[dstv.at[p, i]], add=True)

      pltpu.make_async_copy(src_hbm.at[wid, 0], srcv.at[np_], semi).wait()
      pltpu.make_async_copy(dst_hbm.at[wid, 0], dstv.at[np_], semi).wait()
      return carry

    lax.fori_loop(0, _NSB, superblock, 0)

    plsc.subcore_barrier()
    pltpu.sync_copy(aggsh.at[pl.ds(s * _RPS, _RPS)],
                    agg_out.at[c, pl.ds(s * _RPS, _RPS)])

  return pl.kernel(body, out_type=out_type, mesh=_mesh(),
                   scratch_types=scratch, name="sc_agg")


_EBLK = 4000
_NEB = _E // _EBLK
_HB = _NP // _G


def _hist_body(dst_ref, out_ref):
  i = pl.program_id(0)

  @pl.when(i == 0)
  def _():
    out_ref[...] = jnp.zeros_like(out_ref)

  d = dst_ref[0, 0, :]
  hi = lax.shift_right_logical(d, 7)
  lo = jnp.bitwise_and(d, 127)
  oh_hi = (lax.broadcasted_iota(jnp.int32, (_HB, _EBLK), 0)
           == hi[None, :]).astype(jnp.bfloat16)
  oh_lo = (lax.broadcasted_iota(jnp.int32, (_EBLK, _G), 1)
           == lo[:, None]).astype(jnp.bfloat16)
  out_ref[...] += jnp.dot(oh_hi, oh_lo, preferred_element_type=jnp.float32)


def _hist_tc(dst3):
  return pl.pallas_call(
      _hist_body,
      grid=(_NEB,),
      in_specs=[pl.BlockSpec((1, 1, _EBLK), lambda i: (i, 0, 0))],
      out_specs=pl.BlockSpec((_HB, _G), lambda i: (0, 0)),
      out_shape=jax.ShapeDtypeStruct((_HB, _G), jnp.float32),
  )(dst3)


_sc_cache = {}


def _get_sc(name):
  if name not in _sc_cache:
    _sc_cache[name] = _make_sc_agg()
  return _sc_cache[name]


def _layer_body(aggp_ref, cnt_ref, x_ref, wlt_ref, bl_ref, wrt_ref, h_ref):
  agg = aggp_ref[0] + aggp_ref[1]
  cnt = cnt_ref[...]
  mm = jnp.dot(agg, wlt_ref[...], preferred_element_type=jnp.float32)
  self_mm = jnp.dot(x_ref[...], wrt_ref[...], preferred_element_type=jnp.float32)
  h = mm / jnp.maximum(cnt, 1.0) + bl_ref[...] + self_mm
  h_ref[...] = jnp.maximum(h, 0.0)


def _layer_tc(aggp, cnt, x, wlt, bl, wrt):
  return pl.pallas_call(
      _layer_body,
      grid=(_NB,),
      in_specs=[
          pl.BlockSpec((_NC, _BLK, _D), lambda i: (0, i, 0)),
          pl.BlockSpec((_BLK, 1), lambda i: (i, 0)),
          pl.BlockSpec((_BLK, _D), lambda i: (i, 0)),
          pl.BlockSpec((_D, _D), lambda i: (0, 0)),
          pl.BlockSpec((1, _D), lambda i: (0, 0)),
          pl.BlockSpec((_D, _D), lambda i: (0, 0)),
      ],
      out_specs=pl.BlockSpec((_BLK, _D), lambda i: (i, 0)),
      out_shape=jax.ShapeDtypeStruct((_N, _D), jnp.float32),
  )(aggp, cnt, x, wlt, bl, wrt)


def _final_body(aggp_ref, cnt_ref, h_ref, batch_ref, wlt_ref, bl_ref, wrt_ref,
                wlint_ref, blin_ref, out_ref, pooled_acc, gcnt_acc):
  i = pl.program_id(0)

  @pl.when(i == 0)
  def _():
    pooled_acc[...] = jnp.zeros_like(pooled_acc)
    gcnt_acc[...] = jnp.zeros_like(gcnt_acc)

  agg = aggp_ref[0] + aggp_ref[1]
  cnt = cnt_ref[...]
  mm = jnp.dot(agg, wlt_ref[...], preferred_element_type=jnp.float32)
  self_mm = jnp.dot(h_ref[...], wrt_ref[...], preferred_element_type=jnp.float32)
  h2 = jnp.maximum(mm / jnp.maximum(cnt, 1.0) + bl_ref[...] + self_mm, 0.0)

  b = batch_ref[0, 0, :]
  onehot = (lax.broadcasted_iota(jnp.int32, (_G, _BLK), 0)
            == b[None, :]).astype(jnp.float32)
  pooled_acc[...] += jnp.dot(onehot, h2, preferred_element_type=jnp.float32)
  gcnt_acc[...] += jnp.sum(onehot, axis=1, keepdims=True)

  @pl.when(i == _NB - 1)
  def _():
    pooled = pooled_acc[...] / jnp.maximum(gcnt_acc[...], 1.0)
    out_ref[...] = (jnp.dot(pooled, wlint_ref[...],
                            preferred_element_type=jnp.float32)
                    + blin_ref[...])


def _final_tc(aggp, cnt, h, batch3, wlt, bl, wrt, wlint_pad, blin_pad):
  return pl.pallas_call(
      _final_body,
      grid=(_NB,),
      in_specs=[
          pl.BlockSpec((_NC, _BLK, _D), lambda i: (0, i, 0)),
          pl.BlockSpec((_BLK, 1), lambda i: (i, 0)),
          pl.BlockSpec((_BLK, _D), lambda i: (i, 0)),
          pl.BlockSpec((1, 1, _BLK), lambda i: (i, 0, 0)),
          pl.BlockSpec((_D, _D), lambda i: (0, 0)),
          pl.BlockSpec((1, _D), lambda i: (0, 0)),
          pl.BlockSpec((_D, _D), lambda i: (0, 0)),
          pl.BlockSpec((_D, _D), lambda i: (0, 0)),
          pl.BlockSpec((1, _D), lambda i: (0, 0)),
      ],
      out_specs=pl.BlockSpec((_G, _D), lambda i: (0, 0)),
      out_shape=jax.ShapeDtypeStruct((_G, _D), jnp.float32),
      scratch_shapes=[
          pltpu.VMEM((_G, _D), jnp.float32),
          pltpu.VMEM((_G, 1), jnp.float32),
      ],
  )(aggp, cnt, h, batch3, wlt, bl, wrt, wlint_pad, blin_pad)


def kernel(x, edge_index, batch, Wl1, bl1, Wr1, Wl2, bl2, Wr2, Wlin, blin):
  pad_src = jnp.zeros((_NW, _PADW), jnp.int32)
  pad_dst = jnp.broadcast_to(
      _N + (jnp.arange(_PADW, dtype=jnp.int32) % (_NP - _N)), (_NW, _PADW))
  src = jnp.concatenate([edge_index[0].reshape(_NW, _EPW), pad_src],
                        axis=1).reshape(_NW, _NSB, _SBC, _CH)
  dst = jnp.concatenate([edge_index[1].reshape(_NW, _EPW), pad_dst],
                        axis=1).reshape(_NW, _NSB, _SBC, _CH)
  zeros = jnp.zeros((_NP, _D), jnp.float32)

  hist = _hist_tc(edge_index[1].reshape(_NEB, 1, _EBLK))
  (agg1p,) = _get_sc("agg")(x, src, dst, zeros)
  cnt = hist.reshape(_HB * _G, 1)[:_N]
  h1 = _layer_tc(agg1p, cnt, x, Wl1.T, bl1.reshape(1, _D), Wr1.T)
  (agg2p,) = _get_sc("agg")(h1, src, dst, zeros)

  batch3 = batch.reshape(_NB, 1, _BLK)
  wlint_pad = jnp.pad(Wlin, ((0, _D - _OUT), (0, 0))).T
  blin_pad = jnp.pad(blin, (0, _D - _OUT)).reshape(1, _D)
  out_pad = _final_tc(agg2p, cnt, h1, batch3, Wl2.T, bl2.reshape(1, _D),
                      Wr2.T, wlint_pad, blin_pad)
  return out_pad[:, :_OUT]


# speedup vs baseline: 3.1325x; 1.2700x over previous
import functools

import jax
import jax.numpy as jnp
from jax import lax
from jax.experimental import pallas as pl
from jax.experimental.pallas import tpu as pltpu
from jax.experimental.pallas import tpu_sc as plsc

_N = 10000
_E = 320000
_D = 128
_G = 128
_OUT = 4

_NC = 2
_NS = 16
_NW = _NC * _NS
_EPW = _E // _NW
_CH = 80
_NCH = _EPW // _CH
_NP = 10240
_RPS = _NP // _NS
_NSB = 5
_SBC = _NCH // _NSB

_BLK = 1000
_NB = _N // _BLK


def _mesh():
  return plsc.VectorSubcoreMesh(core_axis_name="c", subcore_axis_name="s",
                                num_cores=_NC, num_subcores=_NS)


def _make_sc_agg():
  out_type = [jax.ShapeDtypeStruct((_NC, _NP, _D), jnp.float32)]
  scratch = [
      pltpu.VMEM((_SBC, _CH), jnp.int32),
      pltpu.VMEM((_SBC, _CH), jnp.int32),
      pltpu.VMEM((_CH, _D), jnp.float32),
      pltpu.VMEM((_CH, _D), jnp.float32),
      pltpu.VMEM((_CH, _D), jnp.float32),
      pltpu.VMEM_SHARED((_NP, _D), jnp.float32),
      pltpu.SemaphoreType.DMA,
      pltpu.SemaphoreType.DMA,
      pltpu.SemaphoreType.DMA,
  ]

  def body(x_hbm, src_hbm, dst_hbm, zeros_hbm, agg_out,
           srcv, dstv, rows0, rows1, rows2, aggsh, sem0, sem1, sem2):
    c = lax.axis_index("c")
    s = lax.axis_index("s")
    wid = c * _NS + s
    pltpu.sync_copy(src_hbm.at[wid, 0], srcv)
    pltpu.sync_copy(dst_hbm.at[wid, 0], dstv)
    pltpu.async_copy(x_hbm.at[srcv.at[0]], rows0, sem0)
    pltpu.async_copy(x_hbm.at[srcv.at[1]], rows1, sem1)
    pltpu.sync_copy(zeros_hbm.at[pl.ds(s * _RPS, _RPS)],
                    aggsh.at[pl.ds(s * _RPS, _RPS)])
    plsc.subcore_barrier()

    def superblock(b, carry):
      @pl.when(b > 0)
      def _():
        pltpu.sync_copy(src_hbm.at[wid, b], srcv)
        pltpu.sync_copy(dst_hbm.at[wid, b], dstv)
        pltpu.async_copy(x_hbm.at[srcv.at[0]], rows0, sem0)
        pltpu.async_copy(x_hbm.at[srcv.at[1]], rows1, sem1)

      def triple(t, carry2):
        base = 3 * t
        pltpu.async_copy(x_hbm.at[srcv.at[base + 2]], rows2, sem2)
        pltpu.make_async_copy(x_hbm.at[srcv.at[0]], rows0, sem0).wait()
        pltpu.sync_copy(rows0, aggsh.at[dstv.at[base]], add=True)

        pltpu.async_copy(x_hbm.at[srcv.at[base + 3]], rows0, sem0)
        pltpu.make_async_copy(x_hbm.at[srcv.at[0]], rows1, sem1).wait()
        pltpu.sync_copy(rows1, aggsh.at[dstv.at[base + 1]], add=True)

        @pl.when(base + 4 < _SBC)
        def _():
          pltpu.async_copy(x_hbm.at[srcv.at[base + 4]], rows1, sem1)

        pltpu.make_async_copy(x_hbm.at[srcv.at[0]], rows2, sem2).wait()
        pltpu.sync_copy(rows2, aggsh.at[dstv.at[base + 2]], add=True)
        return carry2

      lax.fori_loop(0, _SBC // 3, triple, 0)
      pltpu.make_async_copy(x_hbm.at[srcv.at[0]], rows0, sem0).wait()
      pltpu.sync_copy(rows0, aggsh.at[dstv.at[_SBC - 1]], add=True)
      return carry

    lax.fori_loop(0, _NSB, superblock, 0)

    plsc.subcore_barrier()
    pltpu.sync_copy(aggsh.at[pl.ds(s * _RPS, _RPS)],
                    agg_out.at[c, pl.ds(s * _RPS, _RPS)])

  return pl.kernel(body, out_type=out_type, mesh=_mesh(),
                   scratch_types=scratch, name="sc_agg")


_EBLK = 8000
_NEB = _E // _EBLK
_HB = _NP // _G


def _hist_body(dst_ref, out_ref):
  i = pl.program_id(0)

  @pl.when(i == 0)
  def _():
    out_ref[...] = jnp.zeros_like(out_ref)

  d = dst_ref[0, 0, :]
  hi = lax.shift_right_logical(d, 7)
  lo = jnp.bitwise_and(d, 127)
  oh_hi = (lax.broadcasted_iota(jnp.int32, (_HB, _EBLK), 0)
           == hi[None, :]).astype(jnp.bfloat16)
  oh_lo = (lax.broadcasted_iota(jnp.int32, (_EBLK, _G), 1)
           == lo[:, None]).astype(jnp.bfloat16)
  out_ref[...] += jnp.dot(oh_hi, oh_lo, preferred_element_type=jnp.float32)


def _hist_tc(dst3):
  return pl.pallas_call(
      _hist_body,
      grid=(_NEB,),
      in_specs=[pl.BlockSpec((1, 1, _EBLK), lambda i: (i, 0, 0))],
      out_specs=pl.BlockSpec((_HB, _G), lambda i: (0, 0)),
      out_shape=jax.ShapeDtypeStruct((_HB, _G), jnp.float32),
  )(dst3)


_sc_cache = {}


def _get_sc(name):
  if name not in _sc_cache:
    _sc_cache[name] = _make_sc_agg()
  return _sc_cache[name]


def _layer_body(aggp_ref, cnt_ref, x_ref, wlt_ref, bl_ref, wrt_ref, h_ref):
  agg = aggp_ref[0] + aggp_ref[1]
  cnt = cnt_ref[...]
  mm = jnp.dot(agg, wlt_ref[...], preferred_element_type=jnp.float32)
  self_mm = jnp.dot(x_ref[...], wrt_ref[...], preferred_element_type=jnp.float32)
  h = mm / jnp.maximum(cnt, 1.0) + bl_ref[...] + self_mm
  h_ref[...] = jnp.maximum(h, 0.0)


def _layer_tc(aggp, cnt, x, wlt, bl, wrt):
  return pl.pallas_call(
      _layer_body,
      grid=(_NB,),
      in_specs=[
          pl.BlockSpec((_NC, _BLK, _D), lambda i: (0, i, 0)),
          pl.BlockSpec((_BLK, 1), lambda i: (i, 0)),
          pl.BlockSpec((_BLK, _D), lambda i: (i, 0)),
          pl.BlockSpec((_D, _D), lambda i: (0, 0)),
          pl.BlockSpec((1, _D), lambda i: (0, 0)),
          pl.BlockSpec((_D, _D), lambda i: (0, 0)),
      ],
      out_specs=pl.BlockSpec((_BLK, _D), lambda i: (i, 0)),
      out_shape=jax.ShapeDtypeStruct((_N, _D), jnp.float32),
  )(aggp, cnt, x, wlt, bl, wrt)


def _final_body(aggp_ref, cnt_ref, h_ref, batch_ref, wlt_ref, bl_ref, wrt_ref,
                wlint_ref, blin_ref, out_ref, pooled_acc, gcnt_acc):
  i = pl.program_id(0)

  @pl.when(i == 0)
  def _():
    pooled_acc[...] = jnp.zeros_like(pooled_acc)
    gcnt_acc[...] = jnp.zeros_like(gcnt_acc)

  agg = aggp_ref[0] + aggp_ref[1]
  cnt = cnt_ref[...]
  mm = jnp.dot(agg, wlt_ref[...], preferred_element_type=jnp.float32)
  self_mm = jnp.dot(h_ref[...], wrt_ref[...], preferred_element_type=jnp.float32)
  h2 = jnp.maximum(mm / jnp.maximum(cnt, 1.0) + bl_ref[...] + self_mm, 0.0)

  b = batch_ref[0, 0, :]
  onehot = (lax.broadcasted_iota(jnp.int32, (_G, _BLK), 0)
            == b[None, :]).astype(jnp.float32)
  pooled_acc[...] += jnp.dot(onehot, h2, preferred_element_type=jnp.float32)
  gcnt_acc[...] += jnp.sum(onehot, axis=1, keepdims=True)

  @pl.when(i == _NB - 1)
  def _():
    pooled = pooled_acc[...] / jnp.maximum(gcnt_acc[...], 1.0)
    out_ref[...] = (jnp.dot(pooled, wlint_ref[...],
                            preferred_element_type=jnp.float32)
                    + blin_ref[...])


def _final_tc(aggp, cnt, h, batch3, wlt, bl, wrt, wlint_pad, blin_pad):
  return pl.pallas_call(
      _final_body,
      grid=(_NB,),
      in_specs=[
          pl.BlockSpec((_NC, _BLK, _D), lambda i: (0, i, 0)),
          pl.BlockSpec((_BLK, 1), lambda i: (i, 0)),
          pl.BlockSpec((_BLK, _D), lambda i: (i, 0)),
          pl.BlockSpec((1, 1, _BLK), lambda i: (i, 0, 0)),
          pl.BlockSpec((_D, _D), lambda i: (0, 0)),
          pl.BlockSpec((1, _D), lambda i: (0, 0)),
          pl.BlockSpec((_D, _D), lambda i: (0, 0)),
          pl.BlockSpec((_D, _D), lambda i: (0, 0)),
          pl.BlockSpec((1, _D), lambda i: (0, 0)),
      ],
      out_specs=pl.BlockSpec((_G, _D), lambda i: (0, 0)),
      out_shape=jax.ShapeDtypeStruct((_G, _D), jnp.float32),
      scratch_shapes=[
          pltpu.VMEM((_G, _D), jnp.float32),
          pltpu.VMEM((_G, 1), jnp.float32),
      ],
  )(aggp, cnt, h, batch3, wlt, bl, wrt, wlint_pad, blin_pad)


def kernel(x, edge_index, batch, Wl1, bl1, Wr1, Wl2, bl2, Wr2, Wlin, blin):
  src = edge_index[0].reshape(_NW, _NSB, _SBC, _CH)
  dst = edge_index[1].reshape(_NW, _NSB, _SBC, _CH)
  zeros = jnp.zeros((_NP, _D), jnp.float32)

  hist = _hist_tc(edge_index[1].reshape(_NEB, 1, _EBLK))
  (agg1p,) = _get_sc("agg")(x, src, dst, zeros)
  cnt = hist.reshape(_HB * _G, 1)[:_N]
  h1 = _layer_tc(agg1p, cnt, x, Wl1.T, bl1.reshape(1, _D), Wr1.T)
  (agg2p,) = _get_sc("agg")(h1, src, dst, zeros)

  batch3 = batch.reshape(_NB, 1, _BLK)
  wlint_pad = jnp.pad(Wlin, ((0, _D - _OUT), (0, 0))).T
  blin_pad = jnp.pad(blin, (0, _D - _OUT)).reshape(1, _D)
  out_pad = _final_tc(agg2p, cnt, h1, batch3, Wl2.T, bl2.reshape(1, _D),
                      Wr2.T, wlint_pad, blin_pad)
  return out_pad[:, :_OUT]
